# baseline (speedup 1.0000x reference)
import jax
import jax.numpy as jnp
from jax.experimental import pallas as pl
from jax.experimental.pallas import tpu as pltpu


def _body(pid_ref, x_ref, emb_ref, o_ref):
    o_ref[...] = x_ref[...] + emb_ref[0]


def kernel(x, pert_id, embedding_weight):
    n, d = x.shape
    num_p = embedding_weight.shape[0]
    blk = 2000
    emb3 = embedding_weight.reshape(num_p, 1, d)
    pid = jnp.reshape(pert_id, (-1,))[0:1]
    grid_spec = pltpu.PrefetchScalarGridSpec(
        num_scalar_prefetch=1,
        grid=(n // blk,),
        in_specs=[
            pl.BlockSpec((blk, d), lambda i, pid_ref: (i, 0)),
            pl.BlockSpec((1, 1, d), lambda i, pid_ref: (pid_ref[0], 0, 0)),
        ],
        out_specs=pl.BlockSpec((blk, d), lambda i, pid_ref: (i, 0)),
    )
    return pl.pallas_call(
        _body,
        grid_spec=grid_spec,
        out_shape=jax.ShapeDtypeStruct((n, d), x.dtype),
        compiler_params=pltpu.CompilerParams(
            dimension_semantics=("arbitrary",),
        ),
    )(pid, x, emb3)


# speedup vs baseline: 1.5339x; 1.5339x over previous
import jax
import jax.numpy as jnp
from jax.experimental import pallas as pl
from jax.experimental.pallas import tpu as pltpu


def _body(pid_ref, x_ref, emb_ref, o_ref):
    o_ref[...] = x_ref[...] + emb_ref[0]


def kernel(x, pert_id, embedding_weight):
    n, d = x.shape
    num_p = embedding_weight.shape[0]
    blk = 10000
    emb3 = embedding_weight.reshape(num_p, 1, d)
    pid = jnp.reshape(pert_id, (-1,))[0:1]
    grid_spec = pltpu.PrefetchScalarGridSpec(
        num_scalar_prefetch=1,
        grid=(n // blk,),
        in_specs=[
            pl.BlockSpec((blk, d), lambda i, pid_ref: (i, 0)),
            pl.BlockSpec((1, 1, d), lambda i, pid_ref: (pid_ref[0], 0, 0)),
        ],
        out_specs=pl.BlockSpec((blk, d), lambda i, pid_ref: (i, 0)),
    )
    return pl.pallas_call(
        _body,
        grid_spec=grid_spec,
        out_shape=jax.ShapeDtypeStruct((n, d), x.dtype),
        compiler_params=pltpu.CompilerParams(
            dimension_semantics=("arbitrary",),
        ),
    )(pid, x, emb3)


# speedup vs baseline: 1.5951x; 1.0399x over previous
import jax
import jax.numpy as jnp
from jax.experimental import pallas as pl
from jax.experimental.pallas import tpu as pltpu


def _body(pid_ref, x_ref, emb_ref, o_ref):
    o_ref[...] = x_ref[...] + emb_ref[0]


def kernel(x, pert_id, embedding_weight):
    n, d = x.shape
    num_p = embedding_weight.shape[0]
    blk = 20000
    emb3 = embedding_weight.reshape(num_p, 1, d)
    pid = jnp.reshape(pert_id, (-1,))[0:1]
    grid_spec = pltpu.PrefetchScalarGridSpec(
        num_scalar_prefetch=1,
        grid=(n // blk,),
        in_specs=[
            pl.BlockSpec((blk, d), lambda i, pid_ref: (i, 0)),
            pl.BlockSpec((1, 1, d), lambda i, pid_ref: (pid_ref[0], 0, 0)),
        ],
        out_specs=pl.BlockSpec((blk, d), lambda i, pid_ref: (i, 0)),
    )
    return pl.pallas_call(
        _body,
        grid_spec=grid_spec,
        out_shape=jax.ShapeDtypeStruct((n, d), x.dtype),
        compiler_params=pltpu.CompilerParams(
            dimension_semantics=("arbitrary",),
        ),
    )(pid, x, emb3)


# speedup vs baseline: 1.6007x; 1.0035x over previous
import jax
import jax.numpy as jnp
from jax.experimental import pallas as pl
from jax.experimental.pallas import tpu as pltpu


def _body(pid_ref, x_ref, emb_ref, o_ref):
    o_ref[...] = x_ref[...] + emb_ref[0]


def kernel(x, pert_id, embedding_weight):
    n, d = x.shape
    num_p = embedding_weight.shape[0]
    blk = 25000
    emb3 = embedding_weight.reshape(num_p, 1, d)
    pid = jnp.reshape(pert_id, (-1,))[0:1]
    grid_spec = pltpu.PrefetchScalarGridSpec(
        num_scalar_prefetch=1,
        grid=(n // blk,),
        in_specs=[
            pl.BlockSpec((blk, d), lambda i, pid_ref: (i, 0)),
            pl.BlockSpec((1, 1, d), lambda i, pid_ref: (pid_ref[0], 0, 0)),
        ],
        out_specs=pl.BlockSpec((blk, d), lambda i, pid_ref: (i, 0)),
    )
    return pl.pallas_call(
        _body,
        grid_spec=grid_spec,
        out_shape=jax.ShapeDtypeStruct((n, d), x.dtype),
        compiler_params=pltpu.CompilerParams(
            dimension_semantics=("arbitrary",),
        ),
    )(pid, x, emb3)
